# baseline (speedup 1.0000x reference)
import functools

import jax
import jax.numpy as jnp
from jax import lax
from jax.experimental import pallas as pl
from jax.experimental.pallas import tpu as pltpu
from jax.experimental.pallas import tpu_sc as plsc

KNN = 20
EPS_BN = 1e-5
NEG_SLOPE = 0.2
ROWS = 128

_NEG_BIG = -3.0e38


def _knn_proj_body(xt_ref, x_ref, w1m2_ref, w2_ref, idx_ref, a_ref, bv_ref):
    b = pl.program_id(0)
    n_full = xt_ref.shape[1]
    r = x_ref.shape[2]
    xt = xt_ref[0]
    xq = x_ref[0]
    dot = lax.dot_general(xt, xq, (((1,), (0,)), ((), ())),
                          preferred_element_type=jnp.float32)
    sq_all = jnp.sum(xt * xt, axis=1, keepdims=True)
    sq_q = jnp.sum(xq * xq, axis=0, keepdims=True)
    val = (2.0 * dot - sq_q) - sq_all
    iota = lax.broadcasted_iota(jnp.int32, (n_full, r), 0)
    js = []
    for t in range(KNN):
        j = jnp.argmax(val, axis=0).reshape(1, r)
        js.append(j)
        if t < KNN - 1:
            val = jnp.where(iota == j, _NEG_BIG, val)
    idx_ref[0] = jnp.concatenate(js, axis=0) + b * n_full
    a_ref[0] = lax.dot_general(xq, w1m2_ref[...], (((0,), (0,)), ((), ())),
                               preferred_element_type=jnp.float32)
    bv = lax.dot_general(xq, w2_ref[...], (((0,), (0,)), ((), ())),
                         preferred_element_type=jnp.float32)
    bv_ref[0] = jnp.concatenate([bv, jnp.zeros_like(bv)], axis=1)


def _stage1(xt, x, w1m2t, w2t):
    B, N, C = xt.shape
    OUT = w1m2t.shape[1]
    grid = (B, N // ROWS)
    return pl.pallas_call(
        _knn_proj_body,
        grid=grid,
        in_specs=[
            pl.BlockSpec((1, N, C), lambda b, i: (b, 0, 0)),
            pl.BlockSpec((1, C, ROWS), lambda b, i: (b, 0, i)),
            pl.BlockSpec((C, OUT), lambda b, i: (0, 0)),
            pl.BlockSpec((C, OUT), lambda b, i: (0, 0)),
        ],
        out_specs=[
            pl.BlockSpec((1, KNN, ROWS), lambda b, i: (b, 0, i)),
            pl.BlockSpec((1, ROWS, OUT), lambda b, i: (b, i, 0)),
            pl.BlockSpec((1, ROWS, 2 * OUT), lambda b, i: (b, i, 0)),
        ],
        out_shape=[
            jax.ShapeDtypeStruct((B, KNN, N), jnp.int32),
            jax.ShapeDtypeStruct((B, N, OUT), jnp.float32),
            jax.ShapeDtypeStruct((B, N, 2 * OUT), jnp.float32),
        ],
    )(xt, x, w1m2t, w2t)


_SC_P = 16
_SC_IW = 40
_SC_NSUB = _SC_P * KNN // _SC_IW


def _sc_body(idx_hbm, bv_hbm, a_hbm, mx_hbm, mn_hbm, part_hbm,
             idxva, idxvb, rowsva, rowsvb, av, mxb, mnb, partb,
             sema, semb, nc=2, pts_per_w=512):
    wid = lax.axis_index("s") * nc + lax.axis_index("c")
    base0 = wid * pts_per_w
    nchunk = pts_per_w // _SC_P

    for s in range(5):
        for cc in range(4):
            partb[s, pl.ds(cc * 16, 16)] = jnp.zeros((16,), jnp.float32)

    def fetch(c, idxv, rowsv, sem):
        base = pl.multiple_of(base0 + c * _SC_P, _SC_P)
        row0 = pl.multiple_of(base * KNN // _SC_IW, _SC_NSUB)
        pltpu.sync_copy(idx_hbm.at[pl.ds(row0, _SC_NSUB)], idxv)
        for s in range(_SC_NSUB):
            pltpu.async_copy(bv_hbm.at[idxv.at[s]],
                             rowsv.at[pl.ds(s * _SC_IW, _SC_IW)], sem)

    def drain(idxv, rowsv, sem):
        for s in range(_SC_NSUB):
            pltpu.make_async_copy(bv_hbm.at[idxv.at[s]],
                                  rowsv.at[pl.ds(s * _SC_IW, _SC_IW)],
                                  sem).wait()

    def compute(c, rowsv):
        base = pl.multiple_of(base0 + c * _SC_P, _SC_P)
        pltpu.sync_copy(a_hbm.at[pl.ds(base, _SC_P)], av)

        def pt_body(p, inner):
            for cc in range(4):
                sl = pl.ds(cc * 16, 16)
                v = rowsv[p * KNN, sl]
                s1 = v
                s2 = v * v
                mx = v
                mn = v
                for rr in range(1, KNN):
                    v = rowsv[p * KNN + rr, sl]
                    s1 = s1 + v
                    s2 = s2 + v * v
                    mx = jnp.maximum(mx, v)
                    mn = jnp.minimum(mn, v)
                mxb[p, sl] = mx
                mnb[p, sl] = mn
                a = av[p, sl]
                partb[0, sl] = partb[0, sl] + a
                partb[1, sl] = partb[1, sl] + s1
                partb[2, sl] = partb[2, sl] + a * a
                partb[3, sl] = partb[3, sl] + a * s1
                partb[4, sl] = partb[4, sl] + s2
            return inner

        lax.fori_loop(0, _SC_P, pt_body, 0)
        pltpu.sync_copy(mxb, mx_hbm.at[pl.ds(base, _SC_P)])
        pltpu.sync_copy(mnb, mn_hbm.at[pl.ds(base, _SC_P)])

    fetch(0, idxva, rowsva, sema)

    def body2(i, carry):
        c0 = i * 2
        fetch(c0 + 1, idxvb, rowsvb, semb)
        drain(idxva, rowsva, sema)
        compute(c0, rowsva)

        @pl.when(i < nchunk // 2 - 1)
        def _():
            fetch(c0 + 2, idxva, rowsva, sema)

        drain(idxvb, rowsvb, semb)
        compute(c0 + 1, rowsvb)
        return carry

    lax.fori_loop(0, nchunk // 2, body2, 0)
    pltpu.sync_copy(partb, part_hbm.at[wid])


def _stage2(idx2d, bv128, a2):
    BN = a2.shape[0]
    OUT = a2.shape[1]
    info = plsc.get_sparse_core_info()
    nc, ns = info.num_cores, info.num_subcores
    nw = nc * ns
    pts_per_w = BN // nw
    mesh = plsc.VectorSubcoreMesh(core_axis_name="c", subcore_axis_name="s")
    kern = functools.partial(
        pl.kernel,
        mesh=mesh,
        out_type=[
            jax.ShapeDtypeStruct((BN, OUT), jnp.float32),
            jax.ShapeDtypeStruct((BN, OUT), jnp.float32),
            jax.ShapeDtypeStruct((nw, 5, OUT), jnp.float32),
        ],
        scratch_types=[
            pltpu.VMEM((_SC_NSUB, _SC_IW), jnp.int32),
            pltpu.VMEM((_SC_NSUB, _SC_IW), jnp.int32),
            pltpu.VMEM((_SC_P * KNN, 2 * OUT), jnp.float32),
            pltpu.VMEM((_SC_P * KNN, 2 * OUT), jnp.float32),
            pltpu.VMEM((_SC_P, OUT), jnp.float32),
            pltpu.VMEM((_SC_P, OUT), jnp.float32),
            pltpu.VMEM((_SC_P, OUT), jnp.float32),
            pltpu.VMEM((5, OUT), jnp.float32),
            pltpu.SemaphoreType.DMA,
            pltpu.SemaphoreType.DMA,
        ],
    )
    body = functools.partial(_sc_body, nc=nc, pts_per_w=pts_per_w)
    return kern(body)(idx2d, bv128, a2)


def _finalize_body(a_ref, mx_ref, mn_ref, part_ref, g_ref, b_ref, o_ref,
                   total_cnt=1):
    part = part_ref[...]
    sums = jnp.sum(part, axis=0)
    s_a = sums[0:1]
    s_s1 = sums[1:2]
    s_a2 = sums[2:3]
    s_as1 = sums[3:4]
    s_s2 = sums[4:5]
    inv_cnt = jnp.float32(1.0 / total_cnt)
    mean = (KNN * s_a + s_s1) * inv_cnt
    eh2 = (KNN * s_a2 + 2.0 * s_as1 + s_s2) * inv_cnt
    var = eh2 - mean * mean
    inv = lax.rsqrt(var + EPS_BN)
    scale = g_ref[...] * inv
    shift = b_ref[...] - mean * scale
    msel = jnp.where(scale >= 0.0, mx_ref[...], mn_ref[...])
    pre = (a_ref[...] + msel) * scale + shift
    o_ref[...] = jnp.where(pre >= 0.0, pre, NEG_SLOPE * pre)


def _stage3(a2, mx, mn, part, gamma, beta, total_cnt):
    BN, OUT = a2.shape
    NW = part.shape[0]
    RC = 2048
    grid = (BN // RC,)
    return pl.pallas_call(
        functools.partial(_finalize_body, total_cnt=total_cnt),
        grid=grid,
        in_specs=[
            pl.BlockSpec((RC, OUT), lambda i: (i, 0)),
            pl.BlockSpec((RC, OUT), lambda i: (i, 0)),
            pl.BlockSpec((RC, OUT), lambda i: (i, 0)),
            pl.BlockSpec((NW, 5, OUT), lambda i: (0, 0, 0)),
            pl.BlockSpec((1, OUT), lambda i: (0, 0)),
            pl.BlockSpec((1, OUT), lambda i: (0, 0)),
        ],
        out_specs=pl.BlockSpec((RC, OUT), lambda i: (i, 0)),
        out_shape=jax.ShapeDtypeStruct((BN, OUT), jnp.float32),
    )(a2, mx, mn, part, gamma, beta)


def kernel(x, W, gamma, beta):
    B, C, N = x.shape
    OUT = W.shape[0]
    xt = jnp.transpose(x, (0, 2, 1))
    W1 = W[:, :C]
    W2 = W[:, C:]
    w1m2t = jnp.transpose(W1 - W2)
    w2t = jnp.transpose(W2)
    idx_t, A, Bv = _stage1(xt, x, w1m2t, w2t)
    idx2d = jnp.transpose(idx_t, (0, 2, 1)).reshape(-1, _SC_IW)
    a2 = A.reshape(B * N, OUT)
    bv128 = Bv.reshape(B * N, 2 * OUT)
    mx, mn, part = _stage2(idx2d, bv128, a2)
    outf = _stage3(a2, mx, mn, part, gamma.reshape(1, OUT),
                   beta.reshape(1, OUT), B * N * KNN)
    return jnp.transpose(outf.reshape(B, N, OUT), (0, 2, 1))


# speedup vs baseline: 1.3392x; 1.3392x over previous
import functools

import jax
import jax.numpy as jnp
from jax import lax
from jax.experimental import pallas as pl
from jax.experimental.pallas import tpu as pltpu
from jax.experimental.pallas import tpu_sc as plsc

KNN = 20
EPS_BN = 1e-5
NEG_SLOPE = 0.2
ROWS = 256

_NEG_BIG = -3.0e38


def _knn_proj_body(xt_ref, x_ref, w1m2_ref, w2_ref, idx_ref, a_ref, bv_ref):
    b = pl.program_id(0)
    n_full = xt_ref.shape[1]
    r = x_ref.shape[2]
    xt = xt_ref[0]
    xq = x_ref[0]
    dot = lax.dot_general(xt, xq, (((1,), (0,)), ((), ())),
                          preferred_element_type=jnp.float32)
    sq_all = jnp.sum(xt * xt, axis=1, keepdims=True)
    sq_q = jnp.sum(xq * xq, axis=0, keepdims=True)
    val = (2.0 * dot - sq_q) - sq_all
    iota = lax.broadcasted_iota(jnp.int32, (n_full, r), 0)
    js = []
    for t in range(KNN):
        j = jnp.argmax(val, axis=0).reshape(1, r)
        js.append(j)
        if t < KNN - 1:
            val = jnp.where(iota == j, _NEG_BIG, val)
    idx_ref[0] = jnp.concatenate(js, axis=0) + b * n_full
    a_ref[0] = lax.dot_general(xq, w1m2_ref[...], (((0,), (0,)), ((), ())),
                               preferred_element_type=jnp.float32)
    bv = lax.dot_general(xq, w2_ref[...], (((0,), (0,)), ((), ())),
                         preferred_element_type=jnp.float32)
    bv_ref[0] = jnp.concatenate([bv, jnp.zeros_like(bv)], axis=1)


def _stage1(xt, x, w1m2t, w2t):
    B, N, C = xt.shape
    OUT = w1m2t.shape[1]
    grid = (B, N // ROWS)
    return pl.pallas_call(
        _knn_proj_body,
        grid=grid,
        in_specs=[
            pl.BlockSpec((1, N, C), lambda b, i: (b, 0, 0)),
            pl.BlockSpec((1, C, ROWS), lambda b, i: (b, 0, i)),
            pl.BlockSpec((C, OUT), lambda b, i: (0, 0)),
            pl.BlockSpec((C, OUT), lambda b, i: (0, 0)),
        ],
        out_specs=[
            pl.BlockSpec((1, KNN, ROWS), lambda b, i: (b, 0, i)),
            pl.BlockSpec((1, ROWS, OUT), lambda b, i: (b, i, 0)),
            pl.BlockSpec((1, ROWS, 2 * OUT), lambda b, i: (b, i, 0)),
        ],
        out_shape=[
            jax.ShapeDtypeStruct((B, KNN, N), jnp.int32),
            jax.ShapeDtypeStruct((B, N, OUT), jnp.float32),
            jax.ShapeDtypeStruct((B, N, 2 * OUT), jnp.float32),
        ],
        compiler_params=pltpu.CompilerParams(
            dimension_semantics=("parallel", "parallel")),
    )(xt, x, w1m2t, w2t)


_SC_P = 32
_SC_IW = 80
_SC_NSUB = _SC_P * KNN // _SC_IW


def _sc_body(idx_hbm, bv_hbm, a_hbm, mx_hbm, mn_hbm, part_hbm,
             idxv, rowsv, av, mxb, mnb, partb, sem, nc=2, pts_per_w=512):
    wid = lax.axis_index("s") * nc + lax.axis_index("c")
    base0 = wid * pts_per_w
    nchunk = pts_per_w // _SC_P

    for s in range(5):
        for cc in range(4):
            partb[s, pl.ds(cc * 16, 16)] = jnp.zeros((16,), jnp.float32)

    def chunk_body(c, carry):
        base = pl.multiple_of(base0 + c * _SC_P, _SC_P)
        row0 = pl.multiple_of(base * KNN // _SC_IW, _SC_NSUB)
        pltpu.sync_copy(idx_hbm.at[pl.ds(row0, _SC_NSUB)], idxv)
        copies = [
            pltpu.async_copy(bv_hbm.at[idxv.at[s]],
                             rowsv.at[pl.ds(s * _SC_IW, _SC_IW)], sem)
            for s in range(_SC_NSUB)
        ]
        for cp in copies:
            cp.wait()
        pltpu.sync_copy(a_hbm.at[pl.ds(base, _SC_P)], av)

        def pt_body(p, inner):
            for cc in range(4):
                sl = pl.ds(cc * 16, 16)
                v = rowsv[p * KNN, sl]
                s1 = v
                s2 = v * v
                mx = v
                mn = v
                for rr in range(1, KNN):
                    v = rowsv[p * KNN + rr, sl]
                    s1 = s1 + v
                    s2 = s2 + v * v
                    mx = jnp.maximum(mx, v)
                    mn = jnp.minimum(mn, v)
                mxb[p, sl] = mx
                mnb[p, sl] = mn
                a = av[p, sl]
                partb[0, sl] = partb[0, sl] + a
                partb[1, sl] = partb[1, sl] + s1
                partb[2, sl] = partb[2, sl] + a * a
                partb[3, sl] = partb[3, sl] + a * s1
                partb[4, sl] = partb[4, sl] + s2
            return inner

        lax.fori_loop(0, _SC_P, pt_body, 0)
        pltpu.sync_copy(mxb, mx_hbm.at[pl.ds(base, _SC_P)])
        pltpu.sync_copy(mnb, mn_hbm.at[pl.ds(base, _SC_P)])
        return carry

    lax.fori_loop(0, nchunk, chunk_body, 0)
    pltpu.sync_copy(partb, part_hbm.at[wid])


def _stage2(idx2d, bv128, a2):
    BN = a2.shape[0]
    OUT = a2.shape[1]
    info = plsc.get_sparse_core_info()
    nc, ns = info.num_cores, info.num_subcores
    nw = nc * ns
    pts_per_w = BN // nw
    mesh = plsc.VectorSubcoreMesh(core_axis_name="c", subcore_axis_name="s")
    kern = functools.partial(
        pl.kernel,
        mesh=mesh,
        out_type=[
            jax.ShapeDtypeStruct((BN, OUT), jnp.float32),
            jax.ShapeDtypeStruct((BN, OUT), jnp.float32),
            jax.ShapeDtypeStruct((nw, 5, OUT), jnp.float32),
        ],
        scratch_types=[
            pltpu.VMEM((_SC_NSUB, _SC_IW), jnp.int32),
            pltpu.VMEM((_SC_P * KNN, 2 * OUT), jnp.float32),
            pltpu.VMEM((_SC_P, OUT), jnp.float32),
            pltpu.VMEM((_SC_P, OUT), jnp.float32),
            pltpu.VMEM((_SC_P, OUT), jnp.float32),
            pltpu.VMEM((5, OUT), jnp.float32),
            pltpu.SemaphoreType.DMA,
        ],
    )
    body = functools.partial(_sc_body, nc=nc, pts_per_w=pts_per_w)
    return kern(body)(idx2d, bv128, a2)


def _finalize_body(a_ref, mx_ref, mn_ref, part_ref, g_ref, b_ref, o_ref,
                   total_cnt=1):
    part = part_ref[...]
    sums = jnp.sum(part, axis=0)
    s_a = sums[0:1]
    s_s1 = sums[1:2]
    s_a2 = sums[2:3]
    s_as1 = sums[3:4]
    s_s2 = sums[4:5]
    inv_cnt = jnp.float32(1.0 / total_cnt)
    mean = (KNN * s_a + s_s1) * inv_cnt
    eh2 = (KNN * s_a2 + 2.0 * s_as1 + s_s2) * inv_cnt
    var = eh2 - mean * mean
    inv = lax.rsqrt(var + EPS_BN)
    scale = g_ref[...] * inv
    shift = b_ref[...] - mean * scale
    msel = jnp.where(scale >= 0.0, mx_ref[...], mn_ref[...])
    pre = (a_ref[...] + msel) * scale + shift
    o_ref[...] = jnp.where(pre >= 0.0, pre, NEG_SLOPE * pre)


def _stage3(a2, mx, mn, part, gamma, beta, total_cnt):
    BN, OUT = a2.shape
    NW = part.shape[0]
    RC = 2048
    grid = (BN // RC,)
    return pl.pallas_call(
        functools.partial(_finalize_body, total_cnt=total_cnt),
        grid=grid,
        in_specs=[
            pl.BlockSpec((RC, OUT), lambda i: (i, 0)),
            pl.BlockSpec((RC, OUT), lambda i: (i, 0)),
            pl.BlockSpec((RC, OUT), lambda i: (i, 0)),
            pl.BlockSpec((NW, 5, OUT), lambda i: (0, 0, 0)),
            pl.BlockSpec((1, OUT), lambda i: (0, 0)),
            pl.BlockSpec((1, OUT), lambda i: (0, 0)),
        ],
        out_specs=pl.BlockSpec((RC, OUT), lambda i: (i, 0)),
        out_shape=jax.ShapeDtypeStruct((BN, OUT), jnp.float32),
    )(a2, mx, mn, part, gamma, beta)


def kernel(x, W, gamma, beta):
    B, C, N = x.shape
    OUT = W.shape[0]
    xt = jnp.transpose(x, (0, 2, 1))
    W1 = W[:, :C]
    W2 = W[:, C:]
    w1m2t = jnp.transpose(W1 - W2)
    w2t = jnp.transpose(W2)
    idx_t, A, Bv = _stage1(xt, x, w1m2t, w2t)
    idx2d = jnp.transpose(idx_t, (0, 2, 1)).reshape(-1, _SC_IW)
    a2 = A.reshape(B * N, OUT)
    bv128 = Bv.reshape(B * N, 2 * OUT)
    mx, mn, part = _stage2(idx2d, bv128, a2)
    outf = _stage3(a2, mx, mn, part, gamma.reshape(1, OUT),
                   beta.reshape(1, OUT), B * N * KNN)
    return jnp.transpose(outf.reshape(B, N, OUT), (0, 2, 1))


# speedup vs baseline: 1.3942x; 1.0411x over previous
import functools

import jax
import jax.numpy as jnp
from jax import lax
from jax.experimental import pallas as pl
from jax.experimental.pallas import tpu as pltpu
from jax.experimental.pallas import tpu_sc as plsc

KNN = 20
EPS_BN = 1e-5
NEG_SLOPE = 0.2
ROWS = 256

_NEG_BIG = -3.0e38


def _knn_proj_body(xt_ref, x_ref, w1m2_ref, w2_ref, idx_ref, a_ref, bv_ref):
    b = pl.program_id(0)
    n_full = xt_ref.shape[1]
    r = x_ref.shape[2]
    xt = xt_ref[0]
    xq = x_ref[0]
    dot = lax.dot_general(xt, xq, (((1,), (0,)), ((), ())),
                          preferred_element_type=jnp.float32)
    sq_all = jnp.sum(xt * xt, axis=1, keepdims=True)
    sq_q = jnp.sum(xq * xq, axis=0, keepdims=True)
    val = (2.0 * dot - sq_q) - sq_all
    iota = lax.broadcasted_iota(jnp.int32, (n_full, r), 0)
    js = []
    for t in range(KNN):
        j = jnp.argmax(val, axis=0).reshape(1, r)
        js.append(j)
        if t < KNN - 1:
            val = jnp.where(iota == j, _NEG_BIG, val)
    idx_ref[0] = jnp.concatenate(js, axis=0) + b * n_full
    a_ref[0] = lax.dot_general(xq, w1m2_ref[...], (((0,), (0,)), ((), ())),
                               preferred_element_type=jnp.float32)
    bv = lax.dot_general(xq, w2_ref[...], (((0,), (0,)), ((), ())),
                         preferred_element_type=jnp.float32)
    bv_ref[0] = jnp.concatenate([bv, jnp.zeros_like(bv)], axis=1)


def _stage1(xt, x, w1m2t, w2t):
    B, N, C = xt.shape
    OUT = w1m2t.shape[1]
    grid = (B, N // ROWS)
    return pl.pallas_call(
        _knn_proj_body,
        grid=grid,
        in_specs=[
            pl.BlockSpec((1, N, C), lambda b, i: (b, 0, 0)),
            pl.BlockSpec((1, C, ROWS), lambda b, i: (b, 0, i)),
            pl.BlockSpec((C, OUT), lambda b, i: (0, 0)),
            pl.BlockSpec((C, OUT), lambda b, i: (0, 0)),
        ],
        out_specs=[
            pl.BlockSpec((1, KNN, ROWS), lambda b, i: (b, 0, i)),
            pl.BlockSpec((1, ROWS, OUT), lambda b, i: (b, i, 0)),
            pl.BlockSpec((1, ROWS, 2 * OUT), lambda b, i: (b, i, 0)),
        ],
        out_shape=[
            jax.ShapeDtypeStruct((B, KNN, N), jnp.int32),
            jax.ShapeDtypeStruct((B, N, OUT), jnp.float32),
            jax.ShapeDtypeStruct((B, N, 2 * OUT), jnp.float32),
        ],
        compiler_params=pltpu.CompilerParams(
            dimension_semantics=("parallel", "parallel")),
    )(xt, x, w1m2t, w2t)


_SC_P = 32
_SC_IW = 80
_SC_NSUB = _SC_P * KNN // _SC_IW


def _sc_body(idx_hbm, bv_hbm, a_hbm, mx_hbm, mn_hbm, part_hbm,
             idxv, rowsv, av, mxb, mnb, partb, sem, nc=2, pts_per_w=512):
    wid = lax.axis_index("s") * nc + lax.axis_index("c")
    base0 = wid * pts_per_w
    nchunk = pts_per_w // _SC_P

    for s in range(5):
        for cc in range(4):
            partb[s, pl.ds(cc * 16, 16)] = jnp.zeros((16,), jnp.float32)

    def chunk_body(c, carry):
        base = pl.multiple_of(base0 + c * _SC_P, _SC_P)
        row0 = pl.multiple_of(base * KNN // _SC_IW, _SC_NSUB)
        pltpu.sync_copy(idx_hbm.at[pl.ds(row0, _SC_NSUB)], idxv)
        copies = [
            pltpu.async_copy(bv_hbm.at[idxv.at[s]],
                             rowsv.at[pl.ds(s * _SC_IW, _SC_IW)], sem)
            for s in range(_SC_NSUB)
        ]
        for cp in copies:
            cp.wait()
        pltpu.sync_copy(a_hbm.at[pl.ds(base, _SC_P)], av)

        def pt_body(p, inner):
            for cc in range(4):
                sl = pl.ds(cc * 16, 16)
                v = rowsv[p * KNN, sl]
                s1 = v
                s2 = v * v
                mx = v
                mn = v
                for rr in range(1, KNN):
                    v = rowsv[p * KNN + rr, sl]
                    s1 = s1 + v
                    s2 = s2 + v * v
                    mx = jnp.maximum(mx, v)
                    mn = jnp.minimum(mn, v)
                mxb[p, sl] = mx
                mnb[p, sl] = mn
                a = av[p, sl]
                partb[0, sl] = partb[0, sl] + a
                partb[1, sl] = partb[1, sl] + s1
                partb[2, sl] = partb[2, sl] + a * a
                partb[3, sl] = partb[3, sl] + a * s1
                partb[4, sl] = partb[4, sl] + s2
            return inner

        lax.fori_loop(0, _SC_P, pt_body, 0)
        pltpu.sync_copy(mxb, mx_hbm.at[pl.ds(base, _SC_P)])
        pltpu.sync_copy(mnb, mn_hbm.at[pl.ds(base, _SC_P)])
        return carry

    lax.fori_loop(0, nchunk, chunk_body, 0)
    pltpu.sync_copy(partb, part_hbm.at[wid])


def _stage2(idx2d, bv128, a2):
    BN = a2.shape[0]
    OUT = a2.shape[1]
    info = plsc.get_sparse_core_info()
    nc, ns = info.num_cores, info.num_subcores
    nw = nc * ns
    pts_per_w = BN // nw
    mesh = plsc.VectorSubcoreMesh(core_axis_name="c", subcore_axis_name="s")
    kern = functools.partial(
        pl.kernel,
        mesh=mesh,
        out_type=[
            jax.ShapeDtypeStruct((BN, OUT), jnp.float32),
            jax.ShapeDtypeStruct((BN, OUT), jnp.float32),
            jax.ShapeDtypeStruct((nw, 5, OUT), jnp.float32),
        ],
        scratch_types=[
            pltpu.VMEM((_SC_NSUB, _SC_IW), jnp.int32),
            pltpu.VMEM((_SC_P * KNN, 2 * OUT), jnp.float32),
            pltpu.VMEM((_SC_P, OUT), jnp.float32),
            pltpu.VMEM((_SC_P, OUT), jnp.float32),
            pltpu.VMEM((_SC_P, OUT), jnp.float32),
            pltpu.VMEM((5, OUT), jnp.float32),
            pltpu.SemaphoreType.DMA,
        ],
    )
    body = functools.partial(_sc_body, nc=nc, pts_per_w=pts_per_w)
    return kern(body)(idx2d, bv128, a2)


def _finalize_body(a_ref, mx_ref, mn_ref, part_ref, g_ref, b_ref, o_ref,
                   total_cnt=1):
    part = part_ref[...]
    sums = jnp.sum(part, axis=0)
    s_a = sums[0:1]
    s_s1 = sums[1:2]
    s_a2 = sums[2:3]
    s_as1 = sums[3:4]
    s_s2 = sums[4:5]
    inv_cnt = jnp.float32(1.0 / total_cnt)
    mean = (KNN * s_a + s_s1) * inv_cnt
    eh2 = (KNN * s_a2 + 2.0 * s_as1 + s_s2) * inv_cnt
    var = eh2 - mean * mean
    inv = lax.rsqrt(var + EPS_BN)
    scale = g_ref[...] * inv
    shift = b_ref[...] - mean * scale
    msel = jnp.where(scale >= 0.0, mx_ref[...], mn_ref[...])
    pre = (a_ref[...] + msel) * scale + shift
    o_ref[...] = jnp.where(pre >= 0.0, pre, NEG_SLOPE * pre)


def _stage3(a2, mx, mn, part, gamma, beta, total_cnt):
    BN, OUT = a2.shape
    NW = part.shape[0]
    RC = 2048
    grid = (BN // RC,)
    return pl.pallas_call(
        functools.partial(_finalize_body, total_cnt=total_cnt),
        grid=grid,
        in_specs=[
            pl.BlockSpec((RC, OUT), lambda i: (i, 0)),
            pl.BlockSpec((RC, OUT), lambda i: (i, 0)),
            pl.BlockSpec((RC, OUT), lambda i: (i, 0)),
            pl.BlockSpec((NW, 5, OUT), lambda i: (0, 0, 0)),
            pl.BlockSpec((1, OUT), lambda i: (0, 0)),
            pl.BlockSpec((1, OUT), lambda i: (0, 0)),
        ],
        out_specs=pl.BlockSpec((RC, OUT), lambda i: (i, 0)),
        out_shape=jax.ShapeDtypeStruct((BN, OUT), jnp.float32),
    )(a2, mx, mn, part, gamma, beta)


def kernel(x, W, gamma, beta):
    B, C, N = x.shape
    OUT = W.shape[0]
    xt = jnp.transpose(x, (0, 2, 1))
    W1 = W[:, :C]
    W2 = W[:, C:]
    w1m2t = jnp.transpose(W1 - W2)
    w2t = jnp.transpose(W2)
    hb = B // 2
    halves = []
    for h in range(2):
        sl = slice(h * hb, (h + 1) * hb)
        idx_t, A, Bv = _stage1(xt[sl], x[sl], w1m2t, w2t)
        idx2d = jnp.transpose(idx_t, (0, 2, 1)).reshape(-1, _SC_IW)
        a2h = A.reshape(hb * N, OUT)
        bv128 = Bv.reshape(hb * N, 2 * OUT)
        mx, mn, part = _stage2(idx2d, bv128, a2h)
        halves.append((a2h, mx, mn, part))
    a2 = jnp.concatenate([hv[0] for hv in halves], axis=0)
    mx = jnp.concatenate([hv[1] for hv in halves], axis=0)
    mn = jnp.concatenate([hv[2] for hv in halves], axis=0)
    part = jnp.concatenate([hv[3] for hv in halves], axis=0)
    outf = _stage3(a2, mx, mn, part, gamma.reshape(1, OUT),
                   beta.reshape(1, OUT), B * N * KNN)
    return jnp.transpose(outf.reshape(B, N, OUT), (0, 2, 1))


# speedup vs baseline: 1.4230x; 1.0207x over previous
import functools

import jax
import jax.numpy as jnp
from jax import lax
from jax.experimental import pallas as pl
from jax.experimental.pallas import tpu as pltpu
from jax.experimental.pallas import tpu_sc as plsc

KNN = 20
EPS_BN = 1e-5
NEG_SLOPE = 0.2
ROWS = 256

_NEG_BIG = -3.0e38


def _knn_proj_body(xt_ref, x_ref, w1m2_ref, w2_ref, idx_ref, a_ref, bv_ref):
    b = pl.program_id(0)
    n_full = xt_ref.shape[1]
    r = x_ref.shape[2]
    xt = xt_ref[0]
    xq = x_ref[0]
    dot = lax.dot_general(xt, xq, (((1,), (0,)), ((), ())),
                          preferred_element_type=jnp.float32)
    sq_all = jnp.sum(xt * xt, axis=1, keepdims=True)
    sq_q = jnp.sum(xq * xq, axis=0, keepdims=True)
    val = (2.0 * dot - sq_q) - sq_all
    iota = lax.broadcasted_iota(jnp.int32, (n_full, r), 0)
    js = []
    for t in range(KNN):
        j = jnp.argmax(val, axis=0).reshape(1, r)
        js.append(j)
        if t < KNN - 1:
            val = jnp.where(iota == j, _NEG_BIG, val)
    idx_ref[0] = jnp.concatenate(js, axis=0) + b * n_full
    a_ref[0] = lax.dot_general(xq, w1m2_ref[...], (((0,), (0,)), ((), ())),
                               preferred_element_type=jnp.float32)
    bv = lax.dot_general(xq, w2_ref[...], (((0,), (0,)), ((), ())),
                         preferred_element_type=jnp.float32)
    bv_ref[0] = jnp.concatenate([bv, jnp.zeros_like(bv)], axis=1)


def _stage1(xt, x, w1m2t, w2t):
    B, N, C = xt.shape
    OUT = w1m2t.shape[1]
    grid = (B, N // ROWS)
    return pl.pallas_call(
        _knn_proj_body,
        grid=grid,
        in_specs=[
            pl.BlockSpec((1, N, C), lambda b, i: (b, 0, 0)),
            pl.BlockSpec((1, C, ROWS), lambda b, i: (b, 0, i)),
            pl.BlockSpec((C, OUT), lambda b, i: (0, 0)),
            pl.BlockSpec((C, OUT), lambda b, i: (0, 0)),
        ],
        out_specs=[
            pl.BlockSpec((1, KNN, ROWS), lambda b, i: (b, 0, i)),
            pl.BlockSpec((1, ROWS, OUT), lambda b, i: (b, i, 0)),
            pl.BlockSpec((1, ROWS, 2 * OUT), lambda b, i: (b, i, 0)),
        ],
        out_shape=[
            jax.ShapeDtypeStruct((B, KNN, N), jnp.int32),
            jax.ShapeDtypeStruct((B, N, OUT), jnp.float32),
            jax.ShapeDtypeStruct((B, N, 2 * OUT), jnp.float32),
        ],
        compiler_params=pltpu.CompilerParams(
            dimension_semantics=("parallel", "parallel")),
    )(xt, x, w1m2t, w2t)


_SC_P = 32
_SC_IW = 80
_SC_NSUB = _SC_P * KNN // _SC_IW


def _sc_body(idx_hbm, bv_hbm, a_hbm, mx_hbm, mn_hbm, part_hbm,
             idxv, rowsv, av, mxb, mnb, partb, sem, nc=2, pts_per_w=512):
    wid = lax.axis_index("s") * nc + lax.axis_index("c")
    base0 = wid * pts_per_w
    nchunk = pts_per_w // _SC_P

    for s in range(5):
        for cc in range(4):
            partb[s, pl.ds(cc * 16, 16)] = jnp.zeros((16,), jnp.float32)

    def chunk_body(c, carry):
        base = pl.multiple_of(base0 + c * _SC_P, _SC_P)
        row0 = pl.multiple_of(base * KNN // _SC_IW, _SC_NSUB)
        pltpu.sync_copy(idx_hbm.at[pl.ds(row0, _SC_NSUB)], idxv)
        copies = [
            pltpu.async_copy(bv_hbm.at[idxv.at[s]],
                             rowsv.at[pl.ds(s * _SC_IW, _SC_IW)], sem)
            for s in range(_SC_NSUB)
        ]
        for cp in copies:
            cp.wait()
        pltpu.sync_copy(a_hbm.at[pl.ds(base, _SC_P)], av)

        def pt_body(p, inner):
            for cc in range(4):
                sl = pl.ds(cc * 16, 16)
                v = rowsv[p * KNN, sl]
                s1 = v
                s2 = v * v
                mx = v
                mn = v
                for rr in range(1, KNN):
                    v = rowsv[p * KNN + rr, sl]
                    s1 = s1 + v
                    s2 = s2 + v * v
                    mx = jnp.maximum(mx, v)
                    mn = jnp.minimum(mn, v)
                mxb[p, sl] = mx
                mnb[p, sl] = mn
                a = av[p, sl]
                partb[0, sl] = partb[0, sl] + a
                partb[1, sl] = partb[1, sl] + s1
                partb[2, sl] = partb[2, sl] + a * a
                partb[3, sl] = partb[3, sl] + a * s1
                partb[4, sl] = partb[4, sl] + s2
            return inner

        lax.fori_loop(0, _SC_P, pt_body, 0)
        pltpu.sync_copy(mxb, mx_hbm.at[pl.ds(base, _SC_P)])
        pltpu.sync_copy(mnb, mn_hbm.at[pl.ds(base, _SC_P)])
        return carry

    lax.fori_loop(0, nchunk, chunk_body, 0)
    pltpu.sync_copy(partb, part_hbm.at[wid])


def _stage2(idx2d, bv128, a2):
    BN = a2.shape[0]
    OUT = a2.shape[1]
    info = plsc.get_sparse_core_info()
    nc, ns = info.num_cores, info.num_subcores
    nw = nc * ns
    pts_per_w = BN // nw
    mesh = plsc.VectorSubcoreMesh(core_axis_name="c", subcore_axis_name="s")
    kern = functools.partial(
        pl.kernel,
        mesh=mesh,
        out_type=[
            jax.ShapeDtypeStruct((BN, OUT), jnp.float32),
            jax.ShapeDtypeStruct((BN, OUT), jnp.float32),
            jax.ShapeDtypeStruct((nw, 5, OUT), jnp.float32),
        ],
        scratch_types=[
            pltpu.VMEM((_SC_NSUB, _SC_IW), jnp.int32),
            pltpu.VMEM((_SC_P * KNN, 2 * OUT), jnp.float32),
            pltpu.VMEM((_SC_P, OUT), jnp.float32),
            pltpu.VMEM((_SC_P, OUT), jnp.float32),
            pltpu.VMEM((_SC_P, OUT), jnp.float32),
            pltpu.VMEM((5, OUT), jnp.float32),
            pltpu.SemaphoreType.DMA,
        ],
    )
    body = functools.partial(_sc_body, nc=nc, pts_per_w=pts_per_w)
    return kern(body)(idx2d, bv128, a2)


def _finalize_body(a_ref, mx_ref, mn_ref, part_ref, g_ref, b_ref, o_ref,
                   total_cnt=1):
    part = part_ref[...]
    sums = jnp.sum(part, axis=0)
    s_a = sums[0:1]
    s_s1 = sums[1:2]
    s_a2 = sums[2:3]
    s_as1 = sums[3:4]
    s_s2 = sums[4:5]
    inv_cnt = jnp.float32(1.0 / total_cnt)
    mean = (KNN * s_a + s_s1) * inv_cnt
    eh2 = (KNN * s_a2 + 2.0 * s_as1 + s_s2) * inv_cnt
    var = eh2 - mean * mean
    inv = lax.rsqrt(var + EPS_BN)
    scale = g_ref[...] * inv
    shift = b_ref[...] - mean * scale
    msel = jnp.where(scale >= 0.0, mx_ref[...], mn_ref[...])
    pre = (a_ref[...] + msel) * scale + shift
    o_ref[...] = jnp.where(pre >= 0.0, pre, NEG_SLOPE * pre)


def _stage3(a2, mx, mn, part, gamma, beta, total_cnt):
    BN, OUT = a2.shape
    NW = part.shape[0]
    RC = 2048
    grid = (BN // RC,)
    return pl.pallas_call(
        functools.partial(_finalize_body, total_cnt=total_cnt),
        grid=grid,
        in_specs=[
            pl.BlockSpec((RC, OUT), lambda i: (i, 0)),
            pl.BlockSpec((RC, OUT), lambda i: (i, 0)),
            pl.BlockSpec((RC, OUT), lambda i: (i, 0)),
            pl.BlockSpec((NW, 5, OUT), lambda i: (0, 0, 0)),
            pl.BlockSpec((1, OUT), lambda i: (0, 0)),
            pl.BlockSpec((1, OUT), lambda i: (0, 0)),
        ],
        out_specs=pl.BlockSpec((RC, OUT), lambda i: (i, 0)),
        out_shape=jax.ShapeDtypeStruct((BN, OUT), jnp.float32),
    )(a2, mx, mn, part, gamma, beta)


def kernel(x, W, gamma, beta):
    B, C, N = x.shape
    OUT = W.shape[0]
    xt = jnp.transpose(x, (0, 2, 1))
    W1 = W[:, :C]
    W2 = W[:, C:]
    w1m2t = jnp.transpose(W1 - W2)
    w2t = jnp.transpose(W2)
    hb = 1
    halves = []
    for h in range(B):
        sl = slice(h * hb, (h + 1) * hb)
        idx_t, A, Bv = _stage1(xt[sl], x[sl], w1m2t, w2t)
        idx2d = jnp.transpose(idx_t, (0, 2, 1)).reshape(-1, _SC_IW)
        a2h = A.reshape(hb * N, OUT)
        bv128 = Bv.reshape(hb * N, 2 * OUT)
        mx, mn, part = _stage2(idx2d, bv128, a2h)
        halves.append((a2h, mx, mn, part))
    a2 = jnp.concatenate([hv[0] for hv in halves], axis=0)
    mx = jnp.concatenate([hv[1] for hv in halves], axis=0)
    mn = jnp.concatenate([hv[2] for hv in halves], axis=0)
    part = jnp.concatenate([hv[3] for hv in halves], axis=0)
    outf = _stage3(a2, mx, mn, part, gamma.reshape(1, OUT),
                   beta.reshape(1, OUT), B * N * KNN)
    return jnp.transpose(outf.reshape(B, N, OUT), (0, 2, 1))
